# Initial kernel scaffold; baseline (speedup 1.0000x reference)
import jax
import jax.numpy as jnp
from jax.experimental import pallas as pl


def kernel(feat, edge_index, W_self, W_neigh, b_neigh):
    raise NotImplementedError("write your pallas kernel here")


# speedup vs baseline: 5.1690x; 5.1690x over previous
import jax
import jax.numpy as jnp
from jax import lax
from jax.experimental import pallas as pl
from jax.experimental.pallas import tpu as pltpu
from jax.experimental.pallas import tpu_sc as plsc

N_NODES_C = 10000
N_EDGES_C = 320000
D_C = 128
DH = D_C // 2

NC = 2
NS = 16
EPT = N_EDGES_C // NS
CHUNK = 128
NFULL = EPT // CHUNK
TAIL = EPT - NFULL * CHUNK
N_PAD = 10240
ROWS_PER_TILE = N_PAD // NS
DEG_W = 16


def _sc_body(featv_hbm, src2_hbm, dst_hbm, zsum_hbm, zdeg_hbm, ones_hbm,
             out_sum, out_deg,
             src_idx, dst_idx, rows, src_idx_t, dst_idx_t, rows_t, ones_v,
             zdeg_v, acc_sum, acc_deg, sem):
    c = lax.axis_index("c")
    s = lax.axis_index("s")

    pltpu.sync_copy(zsum_hbm, rows)
    pltpu.sync_copy(zdeg_hbm, zdeg_v)
    pltpu.sync_copy(ones_hbm, ones_v)

    @pl.loop(0, ROWS_PER_TILE // CHUNK)
    def _(j):
        r0 = s * ROWS_PER_TILE + j * CHUNK
        pltpu.sync_copy(rows, acc_sum.at[pl.ds(r0, CHUNK)])

        @pl.when(c == 0)
        def _():
            pltpu.sync_copy(zdeg_v, acc_deg.at[pl.ds(r0, CHUNK)])

    plsc.subcore_barrier()

    base0 = s * EPT

    @pl.loop(0, NFULL)
    def _(i):
        base = base0 + i * CHUNK
        pltpu.sync_copy(src2_hbm.at[pl.ds(c * N_EDGES_C + base, CHUNK)], src_idx)
        pltpu.sync_copy(dst_hbm.at[pl.ds(base, CHUNK)], dst_idx)
        pltpu.async_copy(featv_hbm.at[src_idx], rows, sem).wait()
        pltpu.sync_copy(rows, acc_sum.at[dst_idx], add=True)

        @pl.when(c == 0)
        def _():
            pltpu.sync_copy(ones_v, acc_deg.at[dst_idx], add=True)

    tbase = base0 + NFULL * CHUNK
    pltpu.sync_copy(src2_hbm.at[pl.ds(c * N_EDGES_C + tbase, TAIL)], src_idx_t)
    pltpu.sync_copy(dst_hbm.at[pl.ds(tbase, TAIL)], dst_idx_t)
    pltpu.async_copy(featv_hbm.at[src_idx_t], rows_t, sem).wait()
    pltpu.sync_copy(rows_t, acc_sum.at[dst_idx_t], add=True)

    @pl.when(c == 0)
    def _():
        pltpu.sync_copy(ones_v.at[pl.ds(0, TAIL)], acc_deg.at[dst_idx_t],
                        add=True)

    plsc.subcore_barrier()

    @pl.loop(0, ROWS_PER_TILE // CHUNK)
    def _(j):
        r0 = s * ROWS_PER_TILE + j * CHUNK
        pltpu.sync_copy(acc_sum.at[pl.ds(r0, CHUNK)], rows)
        pltpu.sync_copy(rows, out_sum.at[c, pl.ds(r0, CHUNK)])

        @pl.when(c == 0)
        def _():
            pltpu.sync_copy(acc_deg.at[pl.ds(r0, CHUNK)], zdeg_v)
            pltpu.sync_copy(zdeg_v, out_deg.at[pl.ds(r0, CHUNK)])


@jax.jit
def _sc_aggregate(featv, src2, dst, zsum, zdeg, ones):
    mesh = plsc.VectorSubcoreMesh(core_axis_name="c", subcore_axis_name="s")
    k = pl.kernel(
        _sc_body,
        out_type=(
            jax.ShapeDtypeStruct((NC, N_PAD, DH), jnp.float32),
            jax.ShapeDtypeStruct((N_PAD, DEG_W), jnp.float32),
        ),
        mesh=mesh,
        scratch_types=[
            pltpu.VMEM((CHUNK,), jnp.int32),
            pltpu.VMEM((CHUNK,), jnp.int32),
            pltpu.VMEM((CHUNK, DH), jnp.float32),
            pltpu.VMEM((TAIL,), jnp.int32),
            pltpu.VMEM((TAIL,), jnp.int32),
            pltpu.VMEM((TAIL, DH), jnp.float32),
            pltpu.VMEM((CHUNK, DEG_W), jnp.float32),
            pltpu.VMEM((CHUNK, DEG_W), jnp.float32),
            pltpu.VMEM_SHARED((N_PAD, DH), jnp.float32),
            pltpu.VMEM_SHARED((N_PAD, DEG_W), jnp.float32),
            pltpu.SemaphoreType.DMA,
        ],
        compiler_params=pltpu.CompilerParams(use_tc_tiling_on_sc=False),
    )
    return k(featv, src2, dst, zsum, zdeg, ones)


def _tc_body(feat_ref, sum_ref, deg_ref, ws_ref, wn_ref, b_ref, out_ref):
    ssum = jnp.concatenate([sum_ref[0], sum_ref[1]], axis=1)
    deg = deg_ref[:, 0:1]
    h = ssum / jnp.maximum(deg, 1.0)
    out_ref[...] = (
        jnp.dot(feat_ref[...], ws_ref[...], preferred_element_type=jnp.float32)
        + jnp.dot(h, wn_ref[...], preferred_element_type=jnp.float32)
        + b_ref[...]
    )


@jax.jit
def _tc_combine(feat, part_sum, deg, W_self, W_neigh, b2d):
    rb = 2000
    grid = (N_NODES_C // rb,)
    return pl.pallas_call(
        _tc_body,
        grid=grid,
        in_specs=[
            pl.BlockSpec((rb, D_C), lambda i: (i, 0)),
            pl.BlockSpec((NC, rb, DH), lambda i: (0, i, 0)),
            pl.BlockSpec((rb, DEG_W), lambda i: (i, 0)),
            pl.BlockSpec((D_C, D_C), lambda i: (0, 0)),
            pl.BlockSpec((D_C, D_C), lambda i: (0, 0)),
            pl.BlockSpec((1, D_C), lambda i: (0, 0)),
        ],
        out_specs=pl.BlockSpec((rb, D_C), lambda i: (i, 0)),
        out_shape=jax.ShapeDtypeStruct((N_NODES_C, D_C), jnp.float32),
    )(feat, part_sum, deg, W_self, W_neigh, b2d)


def kernel(feat, edge_index, W_self, W_neigh, b_neigh):
    src = edge_index[0].astype(jnp.int32)
    dst = edge_index[1].astype(jnp.int32)
    featv = feat.reshape(2 * N_NODES_C, DH)
    src2 = jnp.concatenate([src * 2, src * 2 + 1])
    zsum = jnp.zeros((CHUNK, DH), jnp.float32)
    zdeg = jnp.zeros((CHUNK, DEG_W), jnp.float32)
    ones = jnp.ones((CHUNK, DEG_W), jnp.float32)
    part_sum, deg = _sc_aggregate(featv, src2, dst, zsum, zdeg, ones)
    b2d = b_neigh.reshape(1, D_C)
    return _tc_combine(feat, part_sum, deg, W_self, W_neigh, b2d)


# speedup vs baseline: 11.0820x; 2.1439x over previous
import jax
import jax.numpy as jnp
from jax import lax
from jax.experimental import pallas as pl
from jax.experimental.pallas import tpu as pltpu
from jax.experimental.pallas import tpu_sc as plsc

N_NODES_C = 10000
N_EDGES_C = 320000
D_C = 128
DH = D_C // 2

NC = 2
NS = 16
EPT = N_EDGES_C // NS
CHUNK = 512
NFULL = EPT // CHUNK
NPAIR = NFULL // 2
TAIL = EPT - NFULL * CHUNK
N_PAD = 10000
ROWS_PER_TILE = N_PAD // NS
WB = 125
DEG_W = 16


def _sc_body(featv_hbm, src2_hbm, dst_hbm, zsum_hbm, zdeg_hbm, ones_hbm,
             out_sum, out_deg,
             src_idx0, dst_idx0, rows0, src_idx1, dst_idx1, rows1,
             src_idx_t, dst_idx_t, rows_t, ones_v, zdeg_v,
             acc_sum, acc_deg,
             sem_g0, sem_g1, sem_s0, sem_s1, sem_d0, sem_d1):
    c = lax.axis_index("c")
    s = lax.axis_index("s")

    pltpu.sync_copy(zsum_hbm, rows0.at[pl.ds(0, WB)])
    pltpu.sync_copy(zdeg_hbm, zdeg_v)
    pltpu.sync_copy(ones_hbm, ones_v)

    @pl.loop(0, ROWS_PER_TILE // WB)
    def _(j):
        r0 = s * ROWS_PER_TILE + j * WB
        pltpu.sync_copy(rows0.at[pl.ds(0, WB)], acc_sum.at[pl.ds(r0, WB)])

        @pl.when(c == 0)
        def _():
            pltpu.sync_copy(zdeg_v, acc_deg.at[pl.ds(r0, WB)])

    plsc.subcore_barrier()

    base0 = s * EPT
    csrc = c * N_EDGES_C

    @pl.loop(0, NPAIR)
    def _(j):
        ba = base0 + (2 * j) * CHUNK
        bb = ba + CHUNK
        pltpu.sync_copy(src2_hbm.at[pl.ds(csrc + ba, CHUNK)], src_idx0)
        pltpu.sync_copy(dst_hbm.at[pl.ds(ba, CHUNK)], dst_idx0)
        g0 = pltpu.async_copy(featv_hbm.at[src_idx0], rows0, sem_g0)
        pltpu.sync_copy(src2_hbm.at[pl.ds(csrc + bb, CHUNK)], src_idx1)
        pltpu.sync_copy(dst_hbm.at[pl.ds(bb, CHUNK)], dst_idx1)
        g1 = pltpu.async_copy(featv_hbm.at[src_idx1], rows1, sem_g1)
        g0.wait()
        s0 = pltpu.async_copy(rows0, acc_sum.at[dst_idx0], sem_s0, add=True)

        @pl.when(c == 0)
        def _():
            pltpu.async_copy(ones_v, acc_deg.at[dst_idx0], sem_d0,
                             add=True).wait()

        g1.wait()
        s1 = pltpu.async_copy(rows1, acc_sum.at[dst_idx1], sem_s1, add=True)

        @pl.when(c == 0)
        def _():
            pltpu.async_copy(ones_v, acc_deg.at[dst_idx1], sem_d1,
                             add=True).wait()

        s0.wait()
        s1.wait()

    bl = base0 + 2 * NPAIR * CHUNK
    pltpu.sync_copy(src2_hbm.at[pl.ds(csrc + bl, CHUNK)], src_idx0)
    pltpu.sync_copy(dst_hbm.at[pl.ds(bl, CHUNK)], dst_idx0)
    pltpu.async_copy(featv_hbm.at[src_idx0], rows0, sem_g0).wait()
    pltpu.sync_copy(rows0, acc_sum.at[dst_idx0], add=True)

    @pl.when(c == 0)
    def _():
        pltpu.sync_copy(ones_v, acc_deg.at[dst_idx0], add=True)

    tbase = base0 + NFULL * CHUNK
    pltpu.sync_copy(src2_hbm.at[pl.ds(csrc + tbase, TAIL)], src_idx_t)
    pltpu.sync_copy(dst_hbm.at[pl.ds(tbase, TAIL)], dst_idx_t)
    pltpu.async_copy(featv_hbm.at[src_idx_t], rows_t, sem_g0).wait()
    pltpu.sync_copy(rows_t, acc_sum.at[dst_idx_t], add=True)

    @pl.when(c == 0)
    def _():
        pltpu.sync_copy(ones_v.at[pl.ds(0, TAIL)], acc_deg.at[dst_idx_t],
                        add=True)

    plsc.subcore_barrier()

    @pl.loop(0, ROWS_PER_TILE // WB)
    def _(j):
        r0 = s * ROWS_PER_TILE + j * WB
        pltpu.sync_copy(acc_sum.at[pl.ds(r0, WB)], rows0.at[pl.ds(0, WB)])
        pltpu.sync_copy(rows0.at[pl.ds(0, WB)], out_sum.at[c, pl.ds(r0, WB)])

        @pl.when(c == 0)
        def _():
            pltpu.sync_copy(acc_deg.at[pl.ds(r0, WB)], zdeg_v)
            pltpu.sync_copy(zdeg_v, out_deg.at[pl.ds(r0, WB)])


@jax.jit
def _sc_aggregate(featv, src2, dst, zsum, zdeg, ones):
    mesh = plsc.VectorSubcoreMesh(core_axis_name="c", subcore_axis_name="s")
    k = pl.kernel(
        _sc_body,
        out_type=(
            jax.ShapeDtypeStruct((NC, N_PAD, DH), jnp.float32),
            jax.ShapeDtypeStruct((N_PAD, DEG_W), jnp.float32),
        ),
        mesh=mesh,
        scratch_types=[
            pltpu.VMEM((CHUNK,), jnp.int32),
            pltpu.VMEM((CHUNK,), jnp.int32),
            pltpu.VMEM((CHUNK, DH), jnp.float32),
            pltpu.VMEM((CHUNK,), jnp.int32),
            pltpu.VMEM((CHUNK,), jnp.int32),
            pltpu.VMEM((CHUNK, DH), jnp.float32),
            pltpu.VMEM((TAIL,), jnp.int32),
            pltpu.VMEM((TAIL,), jnp.int32),
            pltpu.VMEM((TAIL, DH), jnp.float32),
            pltpu.VMEM((CHUNK, DEG_W), jnp.float32),
            pltpu.VMEM((WB, DEG_W), jnp.float32),
            pltpu.VMEM_SHARED((N_PAD, DH), jnp.float32),
            pltpu.VMEM_SHARED((N_PAD, DEG_W), jnp.float32),
            pltpu.SemaphoreType.DMA,
            pltpu.SemaphoreType.DMA,
            pltpu.SemaphoreType.DMA,
            pltpu.SemaphoreType.DMA,
            pltpu.SemaphoreType.DMA,
            pltpu.SemaphoreType.DMA,
        ],
        compiler_params=pltpu.CompilerParams(use_tc_tiling_on_sc=False),
    )
    return k(featv, src2, dst, zsum, zdeg, ones)


def _tc_body(feat_ref, sum_ref, deg_ref, ws_ref, wn_ref, b_ref, out_ref):
    ssum = jnp.concatenate([sum_ref[0], sum_ref[1]], axis=1)
    deg = deg_ref[:, 0:1]
    h = ssum / jnp.maximum(deg, 1.0)
    out_ref[...] = (
        jnp.dot(feat_ref[...], ws_ref[...], preferred_element_type=jnp.float32)
        + jnp.dot(h, wn_ref[...], preferred_element_type=jnp.float32)
        + b_ref[...]
    )


@jax.jit
def _tc_combine(feat, part_sum, deg, W_self, W_neigh, b2d):
    rb = 2000
    grid = (N_NODES_C // rb,)
    return pl.pallas_call(
        _tc_body,
        grid=grid,
        in_specs=[
            pl.BlockSpec((rb, D_C), lambda i: (i, 0)),
            pl.BlockSpec((NC, rb, DH), lambda i: (0, i, 0)),
            pl.BlockSpec((rb, DEG_W), lambda i: (i, 0)),
            pl.BlockSpec((D_C, D_C), lambda i: (0, 0)),
            pl.BlockSpec((D_C, D_C), lambda i: (0, 0)),
            pl.BlockSpec((1, D_C), lambda i: (0, 0)),
        ],
        out_specs=pl.BlockSpec((rb, D_C), lambda i: (i, 0)),
        out_shape=jax.ShapeDtypeStruct((N_NODES_C, D_C), jnp.float32),
    )(feat, part_sum, deg, W_self, W_neigh, b2d)


def kernel(feat, edge_index, W_self, W_neigh, b_neigh):
    src = edge_index[0].astype(jnp.int32)
    dst = edge_index[1].astype(jnp.int32)
    featv = feat.reshape(2 * N_NODES_C, DH)
    src2 = jnp.concatenate([src * 2, src * 2 + 1])
    zsum = jnp.zeros((WB, DH), jnp.float32)
    zdeg = jnp.zeros((WB, DEG_W), jnp.float32)
    ones = jnp.ones((CHUNK, DEG_W), jnp.float32)
    part_sum, deg = _sc_aggregate(featv, src2, dst, zsum, zdeg, ones)
    b2d = b_neigh.reshape(1, D_C)
    return _tc_combine(feat, part_sum, deg, W_self, W_neigh, b2d)
